# baseline (speedup 1.0000x reference)
import jax
import jax.numpy as jnp
from jax import lax
from jax.experimental import pallas as pl
from jax.experimental.pallas import tpu as pltpu
from jax.experimental.pallas import tpu_sc as plsc

N_NODES = 10000
N_EDGES = 320000
NODE_DIM = 128
EDGE_DIM = 16
OUT_DIM = 128
HALF = NODE_DIM // 2

NC = 2
NS = 16
CHUNK = 128
N_PAD = 10112
ROWS_PER_TILE = N_PAD // NS
N_CHUNKS = N_EDGES // CHUNK
CHUNKS_FULL = 160
CHUNKS_LAST = N_CHUNKS - (NS - 1) * CHUNKS_FULL
EDGES_PER_T = CHUNKS_FULL * CHUNK


def _sc_body(nodes_hbm, src_hbm, dst_hbm, edge_hbm,
             sn_out, se_out, cnt_out,
             srcs_v, dsts_v, rows2_v, et2_v, es2_v, ones_v, sn_sh, se_sh,
             cnt_sh,
             sem_g, sem_e, sem_sn, sem_sc):
  c = lax.axis_index("c")
  s = lax.axis_index("s")
  z16 = jnp.zeros((16,), jnp.float32)

  def zrow(r, carry):
    for i in range(HALF // 16):
      rows2_v[0, r, pl.ds(i * 16, 16)] = z16
    es2_v[0, r, pl.ds(0, 16)] = z16
    return carry
  lax.fori_loop(0, CHUNK, zrow, 0)
  for i in range(CHUNK // 16):
    ones_v[pl.ds(i * 16, 16)] = z16

  nfull = ROWS_PER_TILE // CHUNK
  rem = ROWS_PER_TILE % CHUNK
  base = s * ROWS_PER_TILE
  for k in range(nfull):
    pltpu.sync_copy(rows2_v.at[0], sn_sh.at[pl.ds(base + k * CHUNK, CHUNK)])
    pltpu.sync_copy(es2_v.at[0], se_sh.at[pl.ds(base + k * CHUNK, CHUNK)])
  if rem:
    pltpu.sync_copy(rows2_v.at[0, pl.ds(0, rem)],
                    sn_sh.at[pl.ds(base + nfull * CHUNK, rem)])
    pltpu.sync_copy(es2_v.at[0, pl.ds(0, rem)],
                    se_sh.at[pl.ds(base + nfull * CHUNK, rem)])

  @pl.when(s == 0)
  def _():
    def zcnt(k, carry):
      pltpu.sync_copy(ones_v, cnt_sh.at[pl.ds(k * CHUNK, CHUNK)])
      return carry
    lax.fori_loop(0, N_PAD // CHUNK, zcnt, 0)

  for i in range(CHUNK // 16):
    ones_v[pl.ds(i * 16, 16)] = jnp.full((16,), 1.0, jnp.float32)

  @pl.when(s < NS - 1)
  def _():
    pltpu.sync_copy(src_hbm.at[pl.ds(s * CHUNKS_FULL, CHUNKS_FULL)], srcs_v)
    pltpu.sync_copy(dst_hbm.at[pl.ds(s * CHUNKS_FULL, CHUNKS_FULL)], dsts_v)

  @pl.when(s == NS - 1)
  def _():
    pltpu.sync_copy(src_hbm.at[pl.ds((NS - 1) * CHUNKS_FULL, CHUNKS_LAST)],
                    srcs_v.at[pl.ds(0, CHUNKS_LAST)])
    pltpu.sync_copy(dst_hbm.at[pl.ds((NS - 1) * CHUNKS_FULL, CHUNKS_LAST)],
                    dsts_v.at[pl.ds(0, CHUNKS_LAST)])

  nchunks = jnp.where(s < NS - 1, CHUNKS_FULL, CHUNKS_LAST)
  nechunks = (nchunks - c + 1) // 2

  def edge_slice(k):
    return edge_hbm.at[:, s * CHUNKS_FULL + 2 * k + c]

  fidx = lax.iota(jnp.int32, 16)
  tr_hi = lax.shift_right_logical(fidx, 2 + 1)
  tr_lo = lax.bitwise_and(fidx, 7) * CHUNK

  def transpose_chunk(b):
    for e in range(CHUNK):
      col = plsc.load_gather(et2_v.at[b], [tr_hi, tr_lo + e])
      es2_v[b, e, pl.ds(0, 16)] = col

  def xform_row(r):
    for i in range(CHUNK // 16):
      srcs_v[r, pl.ds(i * 16, 16)] = srcs_v[r, pl.ds(i * 16, 16)] * 2 + c

  xform_row(0)
  pltpu.async_copy(nodes_hbm.at[srcs_v.at[0]], rows2_v.at[0], sem_g.at[0])
  pltpu.async_copy(edge_slice(0), et2_v.at[0], sem_e.at[0])

  plsc.subcore_barrier()

  def chunk(j, carry):
    b = lax.rem(j, 2)

    @pl.when(j + 1 < nchunks)
    def _():
      xform_row(j + 1)

      @pl.when(j >= 1)
      def _():
        pltpu.make_async_copy(rows2_v.at[1 - b], sn_sh.at[dsts_v.at[j - 1]],
                              sem_sn.at[1 - b]).wait()

      pltpu.async_copy(nodes_hbm.at[srcs_v.at[j + 1]], rows2_v.at[1 - b],
                       sem_g.at[1 - b])

    @pl.when(j + 1 < nechunks)
    def _():
      @pl.when(j >= 1)
      def _():
        pltpu.make_async_copy(
            es2_v.at[1 - b], se_sh.at[dsts_v.at[2 * (j - 1) + c]],
            sem_sc.at[1 - b]).wait()
        pltpu.make_async_copy(
            ones_v, cnt_sh.at[dsts_v.at[2 * (j - 1) + c]],
            sem_sc.at[1 - b]).wait()

      pltpu.async_copy(edge_slice(j + 1), et2_v.at[1 - b], sem_e.at[1 - b])

    pltpu.make_async_copy(nodes_hbm.at[srcs_v.at[j]], rows2_v.at[b],
                          sem_g.at[b]).wait()
    pltpu.async_copy(rows2_v.at[b], sn_sh.at[dsts_v.at[j]], sem_sn.at[b],
                     add=True)

    @pl.when(j < nechunks)
    def _():
      pltpu.make_async_copy(edge_slice(j), et2_v.at[b], sem_e.at[b]).wait()
      transpose_chunk(b)
      pltpu.async_copy(es2_v.at[b], se_sh.at[dsts_v.at[2 * j + c]],
                       sem_sc.at[b], add=True)
      pltpu.async_copy(ones_v, cnt_sh.at[dsts_v.at[2 * j + c]],
                       sem_sc.at[b], add=True)

    return carry

  lax.fori_loop(0, nchunks, chunk, 0)

  pltpu.make_async_copy(rows2_v.at[0], sn_sh.at[dsts_v.at[0]],
                        sem_sn.at[0]).wait()
  pltpu.make_async_copy(rows2_v.at[1], sn_sh.at[dsts_v.at[0]],
                        sem_sn.at[1]).wait()
  for p in range(2):
    pltpu.make_async_copy(es2_v.at[p], se_sh.at[dsts_v.at[0]],
                          sem_sc.at[p]).wait()
    pltpu.make_async_copy(ones_v, cnt_sh.at[dsts_v.at[0]],
                          sem_sc.at[p]).wait()

  plsc.subcore_barrier()

  sl = pl.ds(base, ROWS_PER_TILE)
  pltpu.sync_copy(sn_sh.at[sl], sn_out.at[c, sl])
  pltpu.sync_copy(se_sh.at[sl], se_out.at[c, sl])

  @pl.when(s == 0)
  def _():
    pltpu.sync_copy(cnt_sh, cnt_out.at[c, 0])


def _run_sc(nodes2, src2, dst2, edge_feat):
  mesh = plsc.VectorSubcoreMesh(
      core_axis_name="c", subcore_axis_name="s", num_cores=NC, num_subcores=NS)
  f32 = jnp.float32
  sc_k = pl.kernel(
      _sc_body,
      out_type=[
          jax.ShapeDtypeStruct((NC, N_PAD, HALF), f32),
          jax.ShapeDtypeStruct((NC, N_PAD, EDGE_DIM), f32),
          jax.ShapeDtypeStruct((NC, 1, N_PAD), f32),
      ],
      mesh=mesh,
      compiler_params=pltpu.CompilerParams(use_tc_tiling_on_sc=False, needs_layout_passes=False),
      scratch_types=[
          pltpu.VMEM((CHUNKS_FULL, CHUNK), jnp.int32),
          pltpu.VMEM((CHUNKS_FULL, CHUNK), jnp.int32),
          pltpu.VMEM((2, CHUNK, HALF), f32),
          pltpu.VMEM((2, 2, 8 * CHUNK), f32),
          pltpu.VMEM((2, CHUNK, EDGE_DIM), f32),
          pltpu.VMEM((CHUNK,), f32),
          pltpu.VMEM_SHARED((N_PAD, HALF), f32),
          pltpu.VMEM_SHARED((N_PAD, EDGE_DIM), f32),
          pltpu.VMEM_SHARED((N_PAD,), f32),
          pltpu.SemaphoreType.DMA((2,)),
          pltpu.SemaphoreType.DMA((2,)),
          pltpu.SemaphoreType.DMA((2,)),
          pltpu.SemaphoreType.DMA((2,)),
      ],
  )
  return sc_k(nodes2, src2, dst2, edge_feat)


def _tc_body(x_ref, sn_ref, se_ref, cnt_ref, wn_ref, bn_ref, we_ref, be_ref,
             wc_ref, bc_ref, o_ref):
  f32 = jnp.float32
  x = x_ref[...]
  sn_lo = sn_ref[0]
  sn_hi = sn_ref[1]
  se = se_ref[0] + se_ref[1]
  cm = cnt_ref[...]
  ones = jnp.ones((NC, OUT_DIM), f32)
  cnt = lax.dot_general(cm, ones, (((0,), (0,)), ((), ())),
                        preferred_element_type=f32)
  wn = wn_ref[...]
  h = jnp.dot(x, wn, preferred_element_type=f32) + bn_ref[...]
  agg_sum = (jnp.dot(sn_lo, wn[0:HALF, :], preferred_element_type=f32)
             + jnp.dot(sn_hi, wn[HALF:NODE_DIM, :], preferred_element_type=f32)
             + jnp.dot(se, we_ref[...], preferred_element_type=f32)
             + cnt * (bn_ref[...] + be_ref[...]))
  agg = agg_sum / jnp.maximum(cnt, 1.0)
  o = (jnp.dot(h, wc_ref[0:OUT_DIM, :], preferred_element_type=f32)
       + jnp.dot(agg, wc_ref[OUT_DIM:2 * OUT_DIM, :],
                 preferred_element_type=f32)
       + bc_ref[...])
  o_ref[...] = o


def _run_tc(node_feat, sn, se, cnt, W_node, b_node, W_edge, b_edge, W_comb,
            b_comb):
  f32 = jnp.float32
  B = 2048
  grid = (pl.cdiv(N_NODES, B),)
  return pl.pallas_call(
      _tc_body,
      grid=grid,
      in_specs=[
          pl.BlockSpec((B, NODE_DIM), lambda i: (i, 0)),
          pl.BlockSpec((NC, B, HALF), lambda i: (0, i, 0)),
          pl.BlockSpec((NC, B, EDGE_DIM), lambda i: (0, i, 0)),
          pl.BlockSpec((NC, B), lambda i: (0, i)),
          pl.BlockSpec((NODE_DIM, OUT_DIM), lambda i: (0, 0)),
          pl.BlockSpec((1, OUT_DIM), lambda i: (0, 0)),
          pl.BlockSpec((EDGE_DIM, OUT_DIM), lambda i: (0, 0)),
          pl.BlockSpec((1, OUT_DIM), lambda i: (0, 0)),
          pl.BlockSpec((2 * OUT_DIM, OUT_DIM), lambda i: (0, 0)),
          pl.BlockSpec((1, OUT_DIM), lambda i: (0, 0)),
      ],
      out_specs=pl.BlockSpec((B, OUT_DIM), lambda i: (i, 0)),
      out_shape=jax.ShapeDtypeStruct((N_NODES, OUT_DIM), f32),
  )(node_feat, sn, se, cnt, W_node, b_node.reshape(1, -1), W_edge,
    b_edge.reshape(1, -1), W_comb, b_comb.reshape(1, -1))


def kernel(node_feat, edge_index, edge_feat, W_node, b_node, W_edge, b_edge,
           W_comb, b_comb):
  i32 = jnp.int32
  src2 = edge_index[0].astype(i32).reshape(N_CHUNKS, CHUNK)
  dst2 = edge_index[1].astype(i32).reshape(N_CHUNKS, CHUNK)
  nodes2 = node_feat.reshape(2 * N_NODES, HALF)

  edge4 = (edge_feat.T.reshape(2, 8, N_CHUNKS, CHUNK)
           .transpose(0, 2, 1, 3).reshape(2, N_CHUNKS, 8 * CHUNK))
  sn, se, cnt = _run_sc(nodes2, src2, dst2, edge4)
  return _run_tc(node_feat, sn, se, cnt.reshape(NC, N_PAD), W_node, b_node,
                 W_edge, b_edge, W_comb, b_comb)


# speedup vs baseline: 1.3098x; 1.3098x over previous
import jax
import jax.numpy as jnp
from jax import lax
from jax.experimental import pallas as pl
from jax.experimental.pallas import tpu as pltpu
from jax.experimental.pallas import tpu_sc as plsc

N_NODES = 10000
N_EDGES = 320000
NODE_DIM = 128
EDGE_DIM = 16
OUT_DIM = 128
HALF = NODE_DIM // 2

NC = 2
NS = 16
CHUNK = 128
N_PAD = 10112
ROWS_PER_TILE = N_PAD // NS
N_CHUNKS = N_EDGES // CHUNK
CHUNKS_FULL = 160
CHUNKS_LAST = N_CHUNKS - (NS - 1) * CHUNKS_FULL
EDGES_PER_T = CHUNKS_FULL * CHUNK


def _sc_body(nodes_hbm, src_hbm, dst_hbm, edge_hbm,
             sn_out, se_out, cnt_out,
             srcs_v, dsts_v, rows2_v, et2_v, es2_v, ones_v, sn_sh, se_sh,
             cnt_sh,
             sem_g, sem_e, sem_sn, sem_sc):
  c = lax.axis_index("c")
  s = lax.axis_index("s")
  z16 = jnp.zeros((16,), jnp.float32)

  def zrow(r, carry):
    for i in range(HALF // 16):
      rows2_v[0, r, pl.ds(i * 16, 16)] = z16
    es2_v[0, r, pl.ds(0, 16)] = z16
    return carry
  lax.fori_loop(0, CHUNK, zrow, 0)
  for i in range(CHUNK // 16):
    ones_v[pl.ds(i * 16, 16)] = z16

  nfull = ROWS_PER_TILE // CHUNK
  rem = ROWS_PER_TILE % CHUNK
  base = s * ROWS_PER_TILE
  for k in range(nfull):
    pltpu.sync_copy(rows2_v.at[0], sn_sh.at[pl.ds(base + k * CHUNK, CHUNK)])
    pltpu.sync_copy(es2_v.at[0], se_sh.at[pl.ds(base + k * CHUNK, CHUNK)])
  if rem:
    pltpu.sync_copy(rows2_v.at[0, pl.ds(0, rem)],
                    sn_sh.at[pl.ds(base + nfull * CHUNK, rem)])
    pltpu.sync_copy(es2_v.at[0, pl.ds(0, rem)],
                    se_sh.at[pl.ds(base + nfull * CHUNK, rem)])

  @pl.when(s == 0)
  def _():
    def zcnt(k, carry):
      pltpu.sync_copy(ones_v, cnt_sh.at[pl.ds(k * CHUNK, CHUNK)])
      return carry
    lax.fori_loop(0, N_PAD // CHUNK, zcnt, 0)

  for i in range(CHUNK // 16):
    ones_v[pl.ds(i * 16, 16)] = jnp.full((16,), 1.0, jnp.float32)

  @pl.when(s < NS - 1)
  def _():
    pltpu.sync_copy(src_hbm.at[pl.ds(s * CHUNKS_FULL, CHUNKS_FULL)], srcs_v)
    pltpu.sync_copy(dst_hbm.at[pl.ds(s * CHUNKS_FULL, CHUNKS_FULL)], dsts_v)

  @pl.when(s == NS - 1)
  def _():
    pltpu.sync_copy(src_hbm.at[pl.ds((NS - 1) * CHUNKS_FULL, CHUNKS_LAST)],
                    srcs_v.at[pl.ds(0, CHUNKS_LAST)])
    pltpu.sync_copy(dst_hbm.at[pl.ds((NS - 1) * CHUNKS_FULL, CHUNKS_LAST)],
                    dsts_v.at[pl.ds(0, CHUNKS_LAST)])

  nchunks = jnp.where(s < NS - 1, CHUNKS_FULL, CHUNKS_LAST)
  nechunks = (nchunks - c + 1) // 2

  def edge_slice(k):
    return edge_hbm.at[:, s * CHUNKS_FULL + 2 * k + c]

  fidx = lax.iota(jnp.int32, 16)
  tr_hi = lax.shift_right_logical(fidx, 3)
  tr_base = lax.bitwise_and(fidx, 7) * CHUNK

  def transpose_chunk(b):
    dst2d = es2_v.at[b]
    src2d = et2_v.at[b]
    for k in range(CHUNK):
      diag = lax.bitwise_and(fidx + k, CHUNK - 1)
      col = plsc.load_gather(src2d, [tr_hi, tr_base + diag])
      plsc.store_scatter(dst2d, [diag, fidx], col)

  def xform_row(r):
    for i in range(CHUNK // 16):
      srcs_v[r, pl.ds(i * 16, 16)] = srcs_v[r, pl.ds(i * 16, 16)] * 2 + c

  xform_row(0)
  pltpu.async_copy(nodes_hbm.at[srcs_v.at[0]], rows2_v.at[0], sem_g.at[0])
  pltpu.async_copy(edge_slice(0), et2_v.at[0], sem_e.at[0])

  plsc.subcore_barrier()

  def chunk(j, carry):
    b = lax.rem(j, 2)

    @pl.when(j + 1 < nchunks)
    def _():
      xform_row(j + 1)

      @pl.when(j >= 1)
      def _():
        pltpu.make_async_copy(rows2_v.at[1 - b], sn_sh.at[dsts_v.at[j - 1]],
                              sem_sn.at[1 - b]).wait()

      pltpu.async_copy(nodes_hbm.at[srcs_v.at[j + 1]], rows2_v.at[1 - b],
                       sem_g.at[1 - b])

    @pl.when(j + 1 < nechunks)
    def _():
      @pl.when(j >= 1)
      def _():
        pltpu.make_async_copy(
            es2_v.at[1 - b], se_sh.at[dsts_v.at[2 * (j - 1) + c]],
            sem_sc.at[1 - b]).wait()
        pltpu.make_async_copy(
            ones_v, cnt_sh.at[dsts_v.at[2 * (j - 1) + c]],
            sem_sc.at[1 - b]).wait()

      pltpu.async_copy(edge_slice(j + 1), et2_v.at[1 - b], sem_e.at[1 - b])

    pltpu.make_async_copy(nodes_hbm.at[srcs_v.at[j]], rows2_v.at[b],
                          sem_g.at[b]).wait()
    pltpu.async_copy(rows2_v.at[b], sn_sh.at[dsts_v.at[j]], sem_sn.at[b],
                     add=True)

    @pl.when(j < nechunks)
    def _():
      pltpu.make_async_copy(edge_slice(j), et2_v.at[b], sem_e.at[b]).wait()
      transpose_chunk(b)
      pltpu.async_copy(es2_v.at[b], se_sh.at[dsts_v.at[2 * j + c]],
                       sem_sc.at[b], add=True)
      pltpu.async_copy(ones_v, cnt_sh.at[dsts_v.at[2 * j + c]],
                       sem_sc.at[b], add=True)

    return carry

  lax.fori_loop(0, nchunks, chunk, 0)

  pltpu.make_async_copy(rows2_v.at[0], sn_sh.at[dsts_v.at[0]],
                        sem_sn.at[0]).wait()
  pltpu.make_async_copy(rows2_v.at[1], sn_sh.at[dsts_v.at[0]],
                        sem_sn.at[1]).wait()
  for p in range(2):
    pltpu.make_async_copy(es2_v.at[p], se_sh.at[dsts_v.at[0]],
                          sem_sc.at[p]).wait()
    pltpu.make_async_copy(ones_v, cnt_sh.at[dsts_v.at[0]],
                          sem_sc.at[p]).wait()

  plsc.subcore_barrier()

  sl = pl.ds(base, ROWS_PER_TILE)
  pltpu.sync_copy(sn_sh.at[sl], sn_out.at[c, sl])
  pltpu.sync_copy(se_sh.at[sl], se_out.at[c, sl])

  @pl.when(s == 0)
  def _():
    pltpu.sync_copy(cnt_sh, cnt_out.at[c, 0])


def _run_sc(nodes2, src2, dst2, edge_feat):
  mesh = plsc.VectorSubcoreMesh(
      core_axis_name="c", subcore_axis_name="s", num_cores=NC, num_subcores=NS)
  f32 = jnp.float32
  sc_k = pl.kernel(
      _sc_body,
      out_type=[
          jax.ShapeDtypeStruct((NC, N_PAD, HALF), f32),
          jax.ShapeDtypeStruct((NC, N_PAD, EDGE_DIM), f32),
          jax.ShapeDtypeStruct((NC, 1, N_PAD), f32),
      ],
      mesh=mesh,
      compiler_params=pltpu.CompilerParams(use_tc_tiling_on_sc=False, needs_layout_passes=False),
      scratch_types=[
          pltpu.VMEM((CHUNKS_FULL, CHUNK), jnp.int32),
          pltpu.VMEM((CHUNKS_FULL, CHUNK), jnp.int32),
          pltpu.VMEM((2, CHUNK, HALF), f32),
          pltpu.VMEM((2, 2, 8 * CHUNK), f32),
          pltpu.VMEM((2, CHUNK, EDGE_DIM), f32),
          pltpu.VMEM((CHUNK,), f32),
          pltpu.VMEM_SHARED((N_PAD, HALF), f32),
          pltpu.VMEM_SHARED((N_PAD, EDGE_DIM), f32),
          pltpu.VMEM_SHARED((N_PAD,), f32),
          pltpu.SemaphoreType.DMA((2,)),
          pltpu.SemaphoreType.DMA((2,)),
          pltpu.SemaphoreType.DMA((2,)),
          pltpu.SemaphoreType.DMA((2,)),
      ],
  )
  return sc_k(nodes2, src2, dst2, edge_feat)


def _tc_body(x_ref, sn_ref, se_ref, cnt_ref, wn_ref, bn_ref, we_ref, be_ref,
             wc_ref, bc_ref, o_ref):
  f32 = jnp.float32
  x = x_ref[...]
  sn_lo = sn_ref[0]
  sn_hi = sn_ref[1]
  se = se_ref[0] + se_ref[1]
  cm = cnt_ref[...]
  ones = jnp.ones((NC, OUT_DIM), f32)
  cnt = lax.dot_general(cm, ones, (((0,), (0,)), ((), ())),
                        preferred_element_type=f32)
  wn = wn_ref[...]
  h = jnp.dot(x, wn, preferred_element_type=f32) + bn_ref[...]
  agg_sum = (jnp.dot(sn_lo, wn[0:HALF, :], preferred_element_type=f32)
             + jnp.dot(sn_hi, wn[HALF:NODE_DIM, :], preferred_element_type=f32)
             + jnp.dot(se, we_ref[...], preferred_element_type=f32)
             + cnt * (bn_ref[...] + be_ref[...]))
  agg = agg_sum / jnp.maximum(cnt, 1.0)
  o = (jnp.dot(h, wc_ref[0:OUT_DIM, :], preferred_element_type=f32)
       + jnp.dot(agg, wc_ref[OUT_DIM:2 * OUT_DIM, :],
                 preferred_element_type=f32)
       + bc_ref[...])
  o_ref[...] = o


def _run_tc(node_feat, sn, se, cnt, W_node, b_node, W_edge, b_edge, W_comb,
            b_comb):
  f32 = jnp.float32
  B = 2048
  grid = (pl.cdiv(N_NODES, B),)
  return pl.pallas_call(
      _tc_body,
      grid=grid,
      in_specs=[
          pl.BlockSpec((B, NODE_DIM), lambda i: (i, 0)),
          pl.BlockSpec((NC, B, HALF), lambda i: (0, i, 0)),
          pl.BlockSpec((NC, B, EDGE_DIM), lambda i: (0, i, 0)),
          pl.BlockSpec((NC, B), lambda i: (0, i)),
          pl.BlockSpec((NODE_DIM, OUT_DIM), lambda i: (0, 0)),
          pl.BlockSpec((1, OUT_DIM), lambda i: (0, 0)),
          pl.BlockSpec((EDGE_DIM, OUT_DIM), lambda i: (0, 0)),
          pl.BlockSpec((1, OUT_DIM), lambda i: (0, 0)),
          pl.BlockSpec((2 * OUT_DIM, OUT_DIM), lambda i: (0, 0)),
          pl.BlockSpec((1, OUT_DIM), lambda i: (0, 0)),
      ],
      out_specs=pl.BlockSpec((B, OUT_DIM), lambda i: (i, 0)),
      out_shape=jax.ShapeDtypeStruct((N_NODES, OUT_DIM), f32),
  )(node_feat, sn, se, cnt, W_node, b_node.reshape(1, -1), W_edge,
    b_edge.reshape(1, -1), W_comb, b_comb.reshape(1, -1))


def kernel(node_feat, edge_index, edge_feat, W_node, b_node, W_edge, b_edge,
           W_comb, b_comb):
  i32 = jnp.int32
  src2 = edge_index[0].astype(i32).reshape(N_CHUNKS, CHUNK)
  dst2 = edge_index[1].astype(i32).reshape(N_CHUNKS, CHUNK)
  nodes2 = node_feat.reshape(2 * N_NODES, HALF)

  edge4 = (edge_feat.T.reshape(2, 8, N_CHUNKS, CHUNK)
           .transpose(0, 2, 1, 3).reshape(2, N_CHUNKS, 8 * CHUNK))
  sn, se, cnt = _run_sc(nodes2, src2, dst2, edge4)
  return _run_tc(node_feat, sn, se, cnt.reshape(NC, N_PAD), W_node, b_node,
                 W_edge, b_edge, W_comb, b_comb)


# speedup vs baseline: 1.4559x; 1.1115x over previous
import jax
import jax.numpy as jnp
from jax import lax
from jax.experimental import pallas as pl
from jax.experimental.pallas import tpu as pltpu
from jax.experimental.pallas import tpu_sc as plsc

N_NODES = 10000
N_EDGES = 320000
NODE_DIM = 128
EDGE_DIM = 16
OUT_DIM = 128
HALF = NODE_DIM // 2

NC = 2
NS = 16
CHUNK = 128
N_PAD = 10112
ROWS_PER_TILE = N_PAD // NS
N_CHUNKS = N_EDGES // CHUNK
CHUNKS_FULL = 160
CHUNKS_LAST = N_CHUNKS - (NS - 1) * CHUNKS_FULL
EDGES_PER_T = CHUNKS_FULL * CHUNK
NBUF = 3


def _sc_body(nodes_hbm, src_hbm, dst_hbm, edge_hbm,
             sn_out, se_out, cnt_out,
             srcs_v, dsts_v, rows2_v, et2_v, es2_v, ones_v, sn_sh, se_sh,
             cnt_sh,
             sem_g, sem_e, sem_sn, sem_sc):
  c = lax.axis_index("c")
  s = lax.axis_index("s")
  z16 = jnp.zeros((16,), jnp.float32)

  def zrow(r, carry):
    for i in range(HALF // 16):
      rows2_v[0, r, pl.ds(i * 16, 16)] = z16
    es2_v[0, r, pl.ds(0, 16)] = z16
    return carry
  lax.fori_loop(0, CHUNK, zrow, 0)
  for i in range(CHUNK // 16):
    ones_v[pl.ds(i * 16, 16)] = z16

  nfull = ROWS_PER_TILE // CHUNK
  rem = ROWS_PER_TILE % CHUNK
  base = s * ROWS_PER_TILE
  for k in range(nfull):
    pltpu.sync_copy(rows2_v.at[0], sn_sh.at[pl.ds(base + k * CHUNK, CHUNK)])
    pltpu.sync_copy(es2_v.at[0], se_sh.at[pl.ds(base + k * CHUNK, CHUNK)])
  if rem:
    pltpu.sync_copy(rows2_v.at[0, pl.ds(0, rem)],
                    sn_sh.at[pl.ds(base + nfull * CHUNK, rem)])
    pltpu.sync_copy(es2_v.at[0, pl.ds(0, rem)],
                    se_sh.at[pl.ds(base + nfull * CHUNK, rem)])

  @pl.when(s == 0)
  def _():
    def zcnt(k, carry):
      pltpu.sync_copy(ones_v, cnt_sh.at[pl.ds(k * CHUNK, CHUNK)])
      return carry
    lax.fori_loop(0, N_PAD // CHUNK, zcnt, 0)

  for i in range(CHUNK // 16):
    ones_v[pl.ds(i * 16, 16)] = jnp.full((16,), 1.0, jnp.float32)

  @pl.when(s < NS - 1)
  def _():
    pltpu.sync_copy(src_hbm.at[pl.ds(s * CHUNKS_FULL, CHUNKS_FULL)], srcs_v)
    pltpu.sync_copy(dst_hbm.at[pl.ds(s * CHUNKS_FULL, CHUNKS_FULL)], dsts_v)

  @pl.when(s == NS - 1)
  def _():
    pltpu.sync_copy(src_hbm.at[pl.ds((NS - 1) * CHUNKS_FULL, CHUNKS_LAST)],
                    srcs_v.at[pl.ds(0, CHUNKS_LAST)])
    pltpu.sync_copy(dst_hbm.at[pl.ds((NS - 1) * CHUNKS_FULL, CHUNKS_LAST)],
                    dsts_v.at[pl.ds(0, CHUNKS_LAST)])

  nchunks = jnp.where(s < NS - 1, CHUNKS_FULL, CHUNKS_LAST)
  nechunks = (nchunks - c + 1) // 2

  def edge_slice(k):
    return edge_hbm.at[:, s * CHUNKS_FULL + 2 * k + c]

  fidx = lax.iota(jnp.int32, 16)
  tr_hi = lax.shift_right_logical(fidx, 3)
  tr_base = lax.bitwise_and(fidx, 7) * CHUNK

  def transpose_chunk(b):
    dst2d = es2_v.at[b]
    src2d = et2_v.at[b]
    for k in range(CHUNK):
      diag = lax.bitwise_and(fidx + k, CHUNK - 1)
      col = plsc.load_gather(src2d, [tr_hi, tr_base + diag])
      plsc.store_scatter(dst2d, [diag, fidx], col)

  def xform_row(r):
    for i in range(CHUNK // 16):
      srcs_v[r, pl.ds(i * 16, 16)] = srcs_v[r, pl.ds(i * 16, 16)] * 2 + c

  xform_row(0)
  xform_row(1)
  pltpu.async_copy(nodes_hbm.at[srcs_v.at[0]], rows2_v.at[0], sem_g.at[0])
  pltpu.async_copy(nodes_hbm.at[srcs_v.at[1]], rows2_v.at[1], sem_g.at[1])
  pltpu.async_copy(edge_slice(0), et2_v.at[0], sem_e.at[0])
  pltpu.async_copy(edge_slice(1), et2_v.at[1], sem_e.at[1])

  plsc.subcore_barrier()

  def chunk(j, carry):
    nb = lax.rem(j, NBUF)
    pf = lax.rem(j + 2, NBUF)

    @pl.when(j + 2 < nchunks)
    def _():
      xform_row(j + 2)

      @pl.when(j >= 1)
      def _():
        pltpu.make_async_copy(rows2_v.at[pf], sn_sh.at[dsts_v.at[j - 1]],
                              sem_sn.at[pf]).wait()

      pltpu.async_copy(nodes_hbm.at[srcs_v.at[j + 2]], rows2_v.at[pf],
                       sem_g.at[pf])

    @pl.when(j + 2 < nechunks)
    def _():
      @pl.when(j >= 1)
      def _():
        pltpu.make_async_copy(
            es2_v.at[pf], se_sh.at[dsts_v.at[2 * (j - 1) + c]],
            sem_sc.at[pf]).wait()
        pltpu.make_async_copy(
            ones_v, cnt_sh.at[dsts_v.at[2 * (j - 1) + c]],
            sem_sc.at[pf]).wait()

      pltpu.async_copy(edge_slice(j + 2), et2_v.at[pf], sem_e.at[pf])

    pltpu.make_async_copy(nodes_hbm.at[srcs_v.at[j]], rows2_v.at[nb],
                          sem_g.at[nb]).wait()
    pltpu.async_copy(rows2_v.at[nb], sn_sh.at[dsts_v.at[j]], sem_sn.at[nb],
                     add=True)

    @pl.when(j < nechunks)
    def _():
      pltpu.make_async_copy(edge_slice(j), et2_v.at[nb], sem_e.at[nb]).wait()
      transpose_chunk(nb)
      pltpu.async_copy(es2_v.at[nb], se_sh.at[dsts_v.at[2 * j + c]],
                       sem_sc.at[nb], add=True)
      pltpu.async_copy(ones_v, cnt_sh.at[dsts_v.at[2 * j + c]],
                       sem_sc.at[nb], add=True)

    return carry

  lax.fori_loop(0, nchunks, chunk, 0)

  def drain(i, carry):
    p = lax.rem(i, NBUF)
    pltpu.make_async_copy(rows2_v.at[p], sn_sh.at[dsts_v.at[0]],
                          sem_sn.at[p]).wait()
    return carry
  lax.fori_loop(nchunks - 3, nchunks, drain, 0)

  def drain_e(i, carry):
    p = lax.rem(i, NBUF)
    pltpu.make_async_copy(es2_v.at[p], se_sh.at[dsts_v.at[0]],
                          sem_sc.at[p]).wait()
    pltpu.make_async_copy(ones_v, cnt_sh.at[dsts_v.at[0]],
                          sem_sc.at[p]).wait()
    return carry
  lax.fori_loop(nechunks - 3, nechunks, drain_e, 0)

  plsc.subcore_barrier()

  sl = pl.ds(base, ROWS_PER_TILE)
  pltpu.sync_copy(sn_sh.at[sl], sn_out.at[c, sl])
  pltpu.sync_copy(se_sh.at[sl], se_out.at[c, sl])

  @pl.when(s == 0)
  def _():
    pltpu.sync_copy(cnt_sh, cnt_out.at[c, 0])


def _run_sc(nodes2, src2, dst2, edge_feat):
  mesh = plsc.VectorSubcoreMesh(
      core_axis_name="c", subcore_axis_name="s", num_cores=NC, num_subcores=NS)
  f32 = jnp.float32
  sc_k = pl.kernel(
      _sc_body,
      out_type=[
          jax.ShapeDtypeStruct((NC, N_PAD, HALF), f32),
          jax.ShapeDtypeStruct((NC, N_PAD, EDGE_DIM), f32),
          jax.ShapeDtypeStruct((NC, 1, N_PAD), f32),
      ],
      mesh=mesh,
      compiler_params=pltpu.CompilerParams(use_tc_tiling_on_sc=False, needs_layout_passes=False),
      scratch_types=[
          pltpu.VMEM((CHUNKS_FULL, CHUNK), jnp.int32),
          pltpu.VMEM((CHUNKS_FULL, CHUNK), jnp.int32),
          pltpu.VMEM((NBUF, CHUNK, HALF), f32),
          pltpu.VMEM((NBUF, 2, 8 * CHUNK), f32),
          pltpu.VMEM((NBUF, CHUNK, EDGE_DIM), f32),
          pltpu.VMEM((CHUNK,), f32),
          pltpu.VMEM_SHARED((N_PAD, HALF), f32),
          pltpu.VMEM_SHARED((N_PAD, EDGE_DIM), f32),
          pltpu.VMEM_SHARED((N_PAD,), f32),
          pltpu.SemaphoreType.DMA((NBUF,)),
          pltpu.SemaphoreType.DMA((NBUF,)),
          pltpu.SemaphoreType.DMA((NBUF,)),
          pltpu.SemaphoreType.DMA((NBUF,)),
      ],
  )
  return sc_k(nodes2, src2, dst2, edge_feat)


def _tc_body(x_ref, sn_ref, se_ref, cnt_ref, wn_ref, bn_ref, we_ref, be_ref,
             wc_ref, bc_ref, o_ref):
  f32 = jnp.float32
  x = x_ref[...]
  sn_lo = sn_ref[0]
  sn_hi = sn_ref[1]
  se = se_ref[0] + se_ref[1]
  cm = cnt_ref[...]
  ones = jnp.ones((NC, OUT_DIM), f32)
  cnt = lax.dot_general(cm, ones, (((0,), (0,)), ((), ())),
                        preferred_element_type=f32)
  wn = wn_ref[...]
  h = jnp.dot(x, wn, preferred_element_type=f32) + bn_ref[...]
  agg_sum = (jnp.dot(sn_lo, wn[0:HALF, :], preferred_element_type=f32)
             + jnp.dot(sn_hi, wn[HALF:NODE_DIM, :], preferred_element_type=f32)
             + jnp.dot(se, we_ref[...], preferred_element_type=f32)
             + cnt * (bn_ref[...] + be_ref[...]))
  agg = agg_sum / jnp.maximum(cnt, 1.0)
  o = (jnp.dot(h, wc_ref[0:OUT_DIM, :], preferred_element_type=f32)
       + jnp.dot(agg, wc_ref[OUT_DIM:2 * OUT_DIM, :],
                 preferred_element_type=f32)
       + bc_ref[...])
  o_ref[...] = o


def _run_tc(node_feat, sn, se, cnt, W_node, b_node, W_edge, b_edge, W_comb,
            b_comb):
  f32 = jnp.float32
  B = 2048
  grid = (pl.cdiv(N_NODES, B),)
  return pl.pallas_call(
      _tc_body,
      grid=grid,
      in_specs=[
          pl.BlockSpec((B, NODE_DIM), lambda i: (i, 0)),
          pl.BlockSpec((NC, B, HALF), lambda i: (0, i, 0)),
          pl.BlockSpec((NC, B, EDGE_DIM), lambda i: (0, i, 0)),
          pl.BlockSpec((NC, B), lambda i: (0, i)),
          pl.BlockSpec((NODE_DIM, OUT_DIM), lambda i: (0, 0)),
          pl.BlockSpec((1, OUT_DIM), lambda i: (0, 0)),
          pl.BlockSpec((EDGE_DIM, OUT_DIM), lambda i: (0, 0)),
          pl.BlockSpec((1, OUT_DIM), lambda i: (0, 0)),
          pl.BlockSpec((2 * OUT_DIM, OUT_DIM), lambda i: (0, 0)),
          pl.BlockSpec((1, OUT_DIM), lambda i: (0, 0)),
      ],
      out_specs=pl.BlockSpec((B, OUT_DIM), lambda i: (i, 0)),
      out_shape=jax.ShapeDtypeStruct((N_NODES, OUT_DIM), f32),
  )(node_feat, sn, se, cnt, W_node, b_node.reshape(1, -1), W_edge,
    b_edge.reshape(1, -1), W_comb, b_comb.reshape(1, -1))


def kernel(node_feat, edge_index, edge_feat, W_node, b_node, W_edge, b_edge,
           W_comb, b_comb):
  i32 = jnp.int32
  src2 = edge_index[0].astype(i32).reshape(N_CHUNKS, CHUNK)
  dst2 = edge_index[1].astype(i32).reshape(N_CHUNKS, CHUNK)
  nodes2 = node_feat.reshape(2 * N_NODES, HALF)

  edge4 = (edge_feat.T.reshape(2, 8, N_CHUNKS, CHUNK)
           .transpose(0, 2, 1, 3).reshape(2, N_CHUNKS, 8 * CHUNK))
  sn, se, cnt = _run_sc(nodes2, src2, dst2, edge4)
  return _run_tc(node_feat, sn, se, cnt.reshape(NC, N_PAD), W_node, b_node,
                 W_edge, b_edge, W_comb, b_comb)
